# baseline (speedup 1.0000x reference)
import functools

import jax
import jax.numpy as jnp
from jax import lax
from jax.experimental import pallas as pl
from jax.experimental.pallas import tpu as pltpu
from jax.experimental.pallas import tpu_sc as plsc

LANE = 128
D = 16
K = 8
NBUF = 2
NUM_WORKERS = 32


def _emb_body(idx_hbm, table_hbm, out_hbm, idx_v, rows_v, sem_g, sem_out):
    n_rows = idx_hbm.shape[0]
    nc = 2
    wid = lax.axis_index("s") * nc + lax.axis_index("c")
    rows_per_tile = n_rows // NUM_WORKERS
    n_chunks = rows_per_tile // K
    base = wid * rows_per_tile

    def run_chunk(r0, b, wait_prev_out):
        if wait_prev_out:
            pltpu.make_async_copy(
                rows_v.at[b], out_hbm.at[pl.ds(r0, K)], sem_out
            ).wait()
        pltpu.sync_copy(idx_hbm.at[pl.ds(r0, K)], idx_v.at[b])
        copies = [
            pltpu.async_copy(
                table_hbm.at[idx_v.at[b].at[j]], rows_v.at[b].at[j], sem_g
            )
            for j in range(K)
        ]
        for c in copies:
            c.wait()
        pltpu.async_copy(rows_v.at[b], out_hbm.at[pl.ds(r0, K)], sem_out)

    for b in range(NBUF):
        run_chunk(base + b * K, b, wait_prev_out=False)

    def body(t, carry):
        for b in range(NBUF):
            run_chunk(base + (t * NBUF + b) * K, b, wait_prev_out=True)
        return carry

    lax.fori_loop(1, n_chunks // NBUF, body, 0)

    for b in range(NBUF):
        pltpu.make_async_copy(
            rows_v.at[b], out_hbm.at[pl.ds(base, K)], sem_out
        ).wait()


def kernel(idxs, embedding_weight):
    b0, b1 = idxs.shape
    n_rows = (b0 * b1) // LANE
    idx2d = idxs.reshape(n_rows, LANE)

    call = functools.partial(
        pl.kernel,
        mesh=plsc.VectorSubcoreMesh(core_axis_name="c", subcore_axis_name="s"),
        out_type=jax.ShapeDtypeStruct((n_rows, LANE, D), jnp.float32),
        scratch_types=[
            pltpu.VMEM((NBUF, K, LANE), jnp.int32),
            pltpu.VMEM((NBUF, K, LANE, D), jnp.float32),
            pltpu.SemaphoreType.DMA,
            pltpu.SemaphoreType.DMA,
        ],
        compiler_params=pltpu.CompilerParams(use_tc_tiling_on_sc=False),
    )(_emb_body)

    out = call(idx2d, embedding_weight)
    return out.reshape(b0, b1, D)


# speedup vs baseline: 2.0035x; 2.0035x over previous
import functools

import jax
import jax.numpy as jnp
from jax import lax
from jax.experimental import pallas as pl
from jax.experimental.pallas import tpu as pltpu
from jax.experimental.pallas import tpu_sc as plsc

D = 16
LANE = 128
NJ = 200
NI = 16384
NUM_WORKERS = 32
IPT = NI // NUM_WORKERS
QT = IPT // LANE


def _emb_body(idx_hbm, table_hbm, out_hbm, idx_v, rows_v, tbuf, sem_i, sem_g, sem_out):
    nc = 2
    wid = lax.axis_index("s") * nc + lax.axis_index("c")
    col0 = wid * IPT
    tc0 = wid * QT
    iota = lax.iota(jnp.int32, 16)

    def stage_idx(j, b):
        pltpu.async_copy(idx_hbm.at[j, pl.ds(col0, IPT)], idx_v.at[b], sem_i)

    def wait_idx(b):
        pltpu.make_async_copy(
            idx_hbm.at[0, pl.ds(col0, IPT)], idx_v.at[b], sem_i
        ).wait()

    def fire_gathers(b):
        for k in range(QT):
            pltpu.async_copy(
                table_hbm.at[idx_v.at[b].at[pl.ds(k * LANE, LANE)]],
                rows_v.at[b].at[pl.ds(k * LANE, LANE)],
                sem_g,
            )

    def wait_gathers(b):
        for k in range(QT):
            pltpu.make_async_copy(
                table_hbm.at[idx_v.at[b].at[pl.ds(k * LANE, LANE)]],
                rows_v.at[b].at[pl.ds(k * LANE, LANE)],
                sem_g,
            ).wait()

    def transpose(b):
        for tr in range(2):
            for r in range(8):
                d = tr * 8 + r
                col_ids = jnp.full((16,), d, jnp.int32)
                for q in range(QT):
                    for g in range(8):
                        row_ids = iota + (q * LANE + g * 16)
                        vals = plsc.load_gather(rows_v.at[b], [row_ids, col_ids])
                        tbuf[b, tr, q, r, pl.ds(g * 16, 16)] = vals

    def writeback(b, j):
        for tr in range(2):
            pltpu.async_copy(
                tbuf.at[b, tr], out_hbm.at[j, tr, pl.ds(tc0, QT)], sem_out
            )

    def wait_writeback(b):
        for tr in range(2):
            pltpu.make_async_copy(
                tbuf.at[b, tr], out_hbm.at[0, tr, pl.ds(tc0, QT)], sem_out
            ).wait()

    stage_idx(0, 0)
    stage_idx(1, 1)
    wait_idx(0)
    fire_gathers(0)

    def body(t, carry):
        for b in range(2):
            j = 2 * t + b
            nb = 1 - b

            @pl.when(j + 1 < NJ)
            def _():
                wait_idx(nb)
                fire_gathers(nb)

            wait_gathers(b)

            @pl.when(j >= 2)
            def _():
                wait_writeback(b)

            @pl.when(j + 2 < NJ)
            def _():
                stage_idx(j + 2, b)

            transpose(b)
            writeback(b, j)
        return carry

    lax.fori_loop(0, NJ // 2, body, 0)
    wait_writeback(0)
    wait_writeback(1)


def kernel(idxs, embedding_weight):
    idx_t = idxs.T

    call = functools.partial(
        pl.kernel,
        mesh=plsc.VectorSubcoreMesh(core_axis_name="c", subcore_axis_name="s"),
        out_type=jax.ShapeDtypeStruct((NJ, 2, NI // LANE, 8, LANE), jnp.float32),
        scratch_types=[
            pltpu.VMEM((2, IPT), jnp.int32),
            pltpu.VMEM((2, IPT, D), jnp.float32),
            pltpu.VMEM((2, 2, QT, 8, LANE), jnp.float32),
            pltpu.SemaphoreType.DMA,
            pltpu.SemaphoreType.DMA,
            pltpu.SemaphoreType.DMA,
        ],
        compiler_params=pltpu.CompilerParams(
            use_tc_tiling_on_sc=False, needs_layout_passes=False
        ),
    )(_emb_body)

    out5d = call(idx_t, embedding_weight)
    return out5d.transpose(2, 4, 0, 1, 3).reshape(NI, NJ, D)
